# baseline (speedup 1.0000x reference)
import jax
import jax.numpy as jnp
from jax import lax
from jax.experimental import pallas as pl
from jax.experimental.pallas import tpu as pltpu

VOCAB = 100000
EMBED = 64
HIDDEN = 128
CTX = 200

BLK = 25000
KBLKS = VOCAB // BLK

NQ = 8


def _fused(idx, emb, W1, b1, W2, b2_blocked):
  def body(idx_ref, emb_ref, w1_ref, b1_ref, w2_ref, b2_ref, out_ref,
           rows_scr, h_scr, ms_scr, sem):
    i = pl.program_id(0)

    @pl.when(i == 0)
    def _():
      copies = []
      for r in range(CTX):
        v = idx_ref[r]
        cp = pltpu.make_async_copy(
            emb_ref.at[pl.ds(v, 1)], rows_scr.at[pl.ds(r, 1)],
            sem.at[r % NQ]
        )
        cp.start()
        copies.append(cp)
      for cp in copies:
        cp.wait()
      ctx = jnp.sum(rows_scr[...], axis=0, keepdims=True)
      h = lax.dot_general(
          ctx, w1_ref[...], (((1,), (1,)), ((), ())),
          preferred_element_type=jnp.float32,
      ) + b1_ref[...]
      h_scr[...] = jnp.maximum(h, 0.0)
      ms_scr[0] = -jnp.inf
      ms_scr[1] = 0.0

    @pl.when(i < KBLKS)
    def _():
      h = h_scr[...]
      logits = lax.dot_general(
          h, w2_ref[...], (((1,), (1,)), ((), ())),
          preferred_element_type=jnp.float32,
      ) + b2_ref[0]
      m = ms_scr[0]
      s = ms_scr[1]
      bm = jnp.max(logits)
      new_m = jnp.maximum(m, bm)
      ms_scr[0] = new_m
      ms_scr[1] = s * jnp.exp(m - new_m) + jnp.sum(jnp.exp(logits - new_m))
      out_ref[pl.ds(i, 1), :] = logits

    @pl.when(i == KBLKS)
    def _():
      lse = ms_scr[0] + jnp.log(ms_scr[1])
      out_ref[...] = out_ref[...] - lse

  return pl.pallas_call(
      body,
      grid=(KBLKS + 1,),
      in_specs=[
          pl.BlockSpec(memory_space=pltpu.SMEM),
          pl.BlockSpec(memory_space=pltpu.MemorySpace.HBM),
          pl.BlockSpec((HIDDEN, EMBED), lambda i: (0, 0)),
          pl.BlockSpec((1, HIDDEN), lambda i: (0, 0)),
          pl.BlockSpec((BLK, HIDDEN), lambda i: (jnp.minimum(i, KBLKS - 1), 0)),
          pl.BlockSpec((1, 1, BLK), lambda i: (jnp.minimum(i, KBLKS - 1), 0, 0)),
      ],
      out_specs=pl.BlockSpec((KBLKS, BLK), lambda i: (0, 0)),
      out_shape=jax.ShapeDtypeStruct((KBLKS, BLK), jnp.float32),
      scratch_shapes=[
          pltpu.VMEM((CTX, EMBED), jnp.float32),
          pltpu.VMEM((1, HIDDEN), jnp.float32),
          pltpu.SMEM((2,), jnp.float32),
          pltpu.SemaphoreType.DMA((NQ,)),
      ],
  )(idx, emb, W1, b1, W2, b2_blocked)


def kernel(inputs, emb, W1, b1, W2, b2):
  idx = inputs.astype(jnp.int32)
  b1r = b1.astype(jnp.float32).reshape(1, HIDDEN)
  b2r = b2.astype(jnp.float32).reshape(KBLKS, 1, BLK)
  out = _fused(idx, emb, W1, b1r, W2, b2r)
  return out.reshape(1, VOCAB)
